# baseline (speedup 1.0000x reference)
import jax
import jax.numpy as jnp
from jax import lax
from jax.experimental import pallas as pl
from jax.experimental.pallas import tpu as pltpu
from jax.experimental.pallas import tpu_sc as plsc

_B, _N, _T = 64, 325, 12
_SPACE_D, _DAY_D, _WEEK_D = 64, 32, 32
_DAY_V, _WEEK_V = 288, 7
_OUT_D = 1 + _SPACE_D + _DAY_D + _WEEK_D
_NP = 8
_ROWS = _NP * _T
_NG = _ROWS // 16
_NCHUNK = -(-_N // _NP)
_LAST_N0 = _N - _NP


def _body(x_hbm, space_hbm, day_hbm, week_hbm, out_hbm,
          x_v, space_v, day_v, week_v, out_v):
    wid = lax.axis_index("s") * 2 + lax.axis_index("c")

    pltpu.sync_copy(space_hbm, space_v)
    pltpu.sync_copy(day_hbm, day_v)
    pltpu.sync_copy(week_hbm, week_v)

    lanes = lax.iota(jnp.int32, 16)
    zeros = lanes * 0
    ones = zeros + 1
    twos = zeros + 2
    c128 = zeros + 128

    def do_chunk(b, n0):
        pltpu.sync_copy(x_hbm.at[b, pl.ds(n0, _NP)], x_v)
        dv, wv = [], []
        for g in range(_NG):
            r = g * 16 + lanes
            n_i = r // _T
            t_i = r - n_i * _T
            fvec = plsc.load_gather(x_v, [n_i, t_i, zeros])
            dvec = plsc.load_gather(x_v, [n_i, t_i, ones]).astype(jnp.int32)
            wvec = plsc.load_gather(x_v, [n_i, t_i, twos]).astype(jnp.int32)
            plsc.store_scatter(out_v, [n_i, t_i, zeros], fvec)
            w31 = plsc.load_gather(week_v, [wvec * _WEEK_D + 31])
            plsc.store_scatter(out_v, [n_i, t_i, c128], w31)
            dv.append(dvec * _DAY_D)
            wv.append(wvec * _WEEK_D)
        for n in range(_NP):
            sb = (n0 + n) * _SPACE_D
            for t in range(_T):
                r = n * _T + t
                g, l = r // 16, r % 16
                db = dv[g][l]
                wb = wv[g][l]
                for k in range(4):
                    out_v[n, t, pl.ds(1 + 16 * k, 16)] = space_v[pl.ds(sb + 16 * k, 16)]
                for k in range(2):
                    out_v[n, t, pl.ds(65 + 16 * k, 16)] = day_v[pl.ds(db + 16 * k, 16)]
                out_v[n, t, pl.ds(97, 16)] = week_v[pl.ds(wb, 16)]
                out_v[n, t, pl.ds(112, 16)] = week_v[pl.ds(wb + 15, 16)]
        pltpu.sync_copy(out_v, out_hbm.at[b, pl.ds(n0, _NP)])

    def chunk(ci, carry):
        n0 = jnp.minimum(ci * _NP, _LAST_N0)
        do_chunk(wid * 2, n0)
        do_chunk(wid * 2 + 1, n0)
        return carry

    lax.fori_loop(0, _NCHUNK, chunk, 0)


def kernel(X, spaceInfo, dayInfo, weekInfo):
    mesh = plsc.VectorSubcoreMesh(core_axis_name="c", subcore_axis_name="s")
    out = pl.kernel(
        _body,
        mesh=mesh,
        compiler_params=pltpu.CompilerParams(
            needs_layout_passes=False, use_tc_tiling_on_sc=False),
        out_type=jax.ShapeDtypeStruct((_B, _N, _T, _OUT_D), jnp.float32),
        scratch_types=[
            pltpu.VMEM((_NP, _T, 3), jnp.float32),
            pltpu.VMEM((_N * _SPACE_D,), jnp.float32),
            pltpu.VMEM((_DAY_V * _DAY_D,), jnp.float32),
            pltpu.VMEM((_WEEK_V * _WEEK_D,), jnp.float32),
            pltpu.VMEM((_NP, _T, _OUT_D), jnp.float32),
        ],
    )(X, spaceInfo.reshape(-1), dayInfo.reshape(-1), weekInfo.reshape(-1))
    return out
